# baseline (speedup 1.0000x reference)
import functools

import jax
import jax.numpy as jnp
from jax.experimental import pallas as pl
from jax.experimental.pallas import tpu as pltpu


def _mixed_kernel(xp_ref, w0_ref, b0_ref, w12_ref, b12_ref,
                  w1_ref, b1_ref, w2_ref, b2_ref, w3_ref, b3_ref, sm_ref,
                  o_ref, *, D, H, W, C1, C2, SP):
    WP = W + 2
    SH = H * WP
    r0 = 2 * WP
    c0n = w0_ref.shape[-1]
    c1n = w1_ref.shape[-1]
    c2n = w2_ref.shape[-1]

    DB = D // 2 if D % 2 == 0 else D

    svalid = (sm_ref[...] != 0)[None, :, :]

    for half in range(D // DB):
        d0 = half * DB
        DS = DB + 2
        M = DB * SH
        m_off = d0 * H * W
        xs = xp_ref[d0:d0 + DS]
        C = xs.shape[-1]

        dd = d0 + jax.lax.broadcasted_iota(jnp.int32, (DS, 1, 1), 0)
        interior = (dd >= 1) & (dd <= D) & svalid

        def emit(y, off, cc):
            y4 = y.reshape(DB, H, WP, cc)[:, :, 1:1 + W, :]
            y2d = y4.reshape(DB * H * W, cc)
            o_ref[off:off + cc, m_off:m_off + DB * H * W] = (
                jnp.transpose(y2d, (1, 0)).astype(o_ref.dtype))

        hid = jnp.dot(xs.reshape(DS * SP, C), w12_ref[...],
                      preferred_element_type=jnp.float32)
        hid = jnp.maximum(hid + b12_ref[...], 0.0)
        hid = jnp.where(interior.reshape(DS * SP, 1), hid, 0.0)
        hs = hid.astype(jnp.bfloat16).reshape(DS, SP, C1 + C2)

        xin = xs[1:1 + DB, r0:r0 + SH, :].reshape(M, C)
        y0 = jnp.maximum(
            jnp.dot(xin, w0_ref[...], preferred_element_type=jnp.float32)
            + b0_ref[...], 0.0)
        emit(y0, 0, c0n)

        acc1 = jnp.zeros((M, c1n), jnp.float32)
        for kd in range(3):
            taps = [hs[kd:kd + DB,
                       r0 + (kh - 1) * WP + kw - 1:
                       r0 + (kh - 1) * WP + kw - 1 + SH, :C1]
                    for kh in range(3) for kw in range(3)]
            wide = jnp.concatenate(taps, axis=-1).reshape(M, 9 * C1)
            acc1 = acc1 + jnp.dot(wide, w1_ref[kd],
                                  preferred_element_type=jnp.float32)
        y1 = jnp.maximum(acc1 + b1_ref[...], 0.0)
        emit(y1, c0n, c1n)

        taps2 = [hs[kd:kd + DB,
                    r0 + (kh - 1) * WP + kw - 1:
                    r0 + (kh - 1) * WP + kw - 1 + SH, C1:]
                 for kd in range(3) for kh in range(3) for kw in range(3)]
        wide2 = jnp.concatenate(taps2, axis=-1).reshape(M, 27 * C2)
        y2 = jnp.maximum(
            jnp.dot(wide2, w2_ref[...], preferred_element_type=jnp.float32)
            + b2_ref[...], 0.0)
        emit(y2, c0n + c1n, c2n)

        mw = jnp.maximum(jnp.maximum(xs[:, 0:SP - 2, :], xs[:, 1:SP - 1, :]),
                         xs[:, 2:SP, :])
        k0 = r0 - WP - 1
        mhc = jnp.maximum(
            jnp.maximum(mw[:, k0:k0 + SH, :], mw[:, k0 + WP:k0 + WP + SH, :]),
            mw[:, k0 + 2 * WP:k0 + 2 * WP + SH, :])
        pooled = jnp.maximum(jnp.maximum(mhc[0:DB], mhc[1:1 + DB]),
                             mhc[2:2 + DB])
        y3 = jnp.maximum(
            jnp.dot(pooled.reshape(M, C), w3_ref[...],
                    preferred_element_type=jnp.float32) + b3_ref[...], 0.0)
        emit(y3, c0n + c1n + c2n, w3_ref.shape[-1])


def kernel(x,
           b0_w, b0_s, b0_b,
           b1a_w, b1a_s, b1a_b,
           b1b_w, b1b_s, b1b_b,
           b2a_w, b2a_s, b2a_b,
           b2b_w, b2b_s, b2b_b,
           b3_w, b3_s, b3_b):
    n, c, d, h, w = x.shape
    bf = jnp.bfloat16
    dp, wp = d + 2, w + 2
    sp = (h + 4) * wp

    xt = jnp.transpose(x, (0, 2, 3, 4, 1)).astype(bf)
    xf = jnp.pad(xt, ((0, 0), (1, 1), (2, 2), (1, 1), (0, 0)),
                 constant_values=-jnp.inf)
    xf = xf.reshape(n, dp, sp, c)

    ii = jnp.arange(sp)
    hh = ii // wp - 1
    ww = ii % wp
    smask = ((hh >= 1) & (hh <= h) & (ww >= 1) & (ww <= w)
             ).astype(jnp.float32).reshape(sp, 1)

    c0 = b0_w.shape[1]
    c1 = b1a_w.shape[1]
    c2 = b2a_w.shape[1]
    c1b = b1b_w.shape[-1]
    c2b = b2b_w.shape[-1]
    c3 = b3_w.shape[1]
    couts = c0 + c1b + c2b + c3

    m = d * h * w

    w0f = (b0_w * b0_s[None, :]).astype(bf)
    w12 = jnp.concatenate([b1a_w * b1a_s[None, :],
                           b2a_w * b2a_s[None, :]], axis=1).astype(bf)
    b12 = jnp.concatenate([b1a_b, b2a_b]).reshape(1, c1 + c2)
    w1f = (b1b_w * b1b_s).reshape(3, 9 * c1, c1b).astype(bf)
    w2f = (b2b_w * b2b_s).reshape(27 * c2, c2b).astype(bf)
    w3f = (b3_w * b3_s[None, :]).astype(bf)

    out = pl.pallas_call(
        functools.partial(_mixed_kernel, D=d, H=h, W=w, C1=c1, C2=c2, SP=sp),
        out_shape=jax.ShapeDtypeStruct((n, couts, d * h * w), jnp.float32),
        grid_spec=pltpu.PrefetchScalarGridSpec(
            num_scalar_prefetch=0,
            grid=(n, 1),
            in_specs=[
                pl.BlockSpec((pl.Squeezed(), dp, sp, c),
                             lambda ni, di: (ni, 0, 0, 0)),
                pl.BlockSpec((c, c0), lambda ni, di: (0, 0)),
                pl.BlockSpec((1, c0), lambda ni, di: (0, 0)),
                pl.BlockSpec((c, c1 + c2), lambda ni, di: (0, 0)),
                pl.BlockSpec((1, c1 + c2), lambda ni, di: (0, 0)),
                pl.BlockSpec((3, 9 * c1, c1b), lambda ni, di: (0, 0, 0)),
                pl.BlockSpec((1, c1b), lambda ni, di: (0, 0)),
                pl.BlockSpec((27 * c2, c2b), lambda ni, di: (0, 0)),
                pl.BlockSpec((1, c2b), lambda ni, di: (0, 0)),
                pl.BlockSpec((c, c3), lambda ni, di: (0, 0)),
                pl.BlockSpec((1, c3), lambda ni, di: (0, 0)),
                pl.BlockSpec((sp, 1), lambda ni, di: (0, 0)),
            ],
            out_specs=pl.BlockSpec((pl.Squeezed(), couts, m),
                                   lambda ni, di: (ni, 0, di)),
        ),
        compiler_params=pltpu.CompilerParams(
            dimension_semantics=("parallel", "parallel"),
            vmem_limit_bytes=60 * 1024 * 1024,
        ),
    )(xf, w0f, b0_b.reshape(1, c0), w12, b12,
      w1f, b1b_b.reshape(1, c1b), w2f, b2b_b.reshape(1, c2b),
      w3f, b3_b.reshape(1, c3), smask)
    return out.reshape(n, couts, d, h, w)


# speedup vs baseline: 1.4547x; 1.4547x over previous
import functools

import jax
import jax.numpy as jnp
from jax.experimental import pallas as pl
from jax.experimental.pallas import tpu as pltpu


def _mixed_kernel(xp_ref, w0_ref, b0_ref, w12_ref, b12_ref,
                  w1_ref, b1_ref, w2_ref, b2_ref, w3_ref, b3_ref, sm_ref,
                  o_ref, *, D, H, W, C1, C2, SP):
    WP = W + 2
    SH = H * WP
    r0 = 2 * WP
    DB = o_ref.shape[0]
    DS = DB + 2
    M = DB * SH
    d0 = pl.multiple_of(pl.program_id(1) * DB, DB)
    xs = xp_ref[pl.ds(d0, DS)]
    C = xs.shape[-1]

    dd = d0 + jax.lax.broadcasted_iota(jnp.int32, (DS, 1, 1), 0)
    svalid = (sm_ref[...] != 0)[None, :, :]
    interior = (dd >= 1) & (dd <= D) & svalid

    hid = jnp.dot(xs.reshape(DS * SP, C), w12_ref[...],
                  preferred_element_type=jnp.float32)
    hid = jnp.maximum(hid + b12_ref[...], 0.0)
    hid = jnp.where(interior.reshape(DS * SP, 1), hid, 0.0)
    hs = hid.astype(jnp.bfloat16).reshape(DS, SP, C1 + C2)

    xin = xs[1:1 + DB, r0:r0 + SH, :].reshape(M, C)
    y0 = jnp.maximum(
        jnp.dot(xin, w0_ref[...], preferred_element_type=jnp.float32)
        + b0_ref[...], 0.0)

    acc1 = jnp.zeros((M, w1_ref.shape[-1]), jnp.float32)
    for kd in range(3):
        taps = [hs[kd:kd + DB,
                   r0 + (kh - 1) * WP + kw - 1:
                   r0 + (kh - 1) * WP + kw - 1 + SH, :C1]
                for kh in range(3) for kw in range(3)]
        wide = jnp.concatenate(taps, axis=-1).reshape(M, 9 * C1)
        acc1 = acc1 + jnp.dot(wide, w1_ref[kd],
                              preferred_element_type=jnp.float32)
    y1 = jnp.maximum(acc1 + b1_ref[...], 0.0)

    taps2 = [hs[kd:kd + DB,
                r0 + (kh - 1) * WP + kw - 1:
                r0 + (kh - 1) * WP + kw - 1 + SH, C1:]
             for kd in range(3) for kh in range(3) for kw in range(3)]
    wide2 = jnp.concatenate(taps2, axis=-1).reshape(M, 27 * C2)
    y2 = jnp.maximum(
        jnp.dot(wide2, w2_ref[...], preferred_element_type=jnp.float32)
        + b2_ref[...], 0.0)

    mw = jnp.maximum(jnp.maximum(xs[:, 0:SP - 2, :], xs[:, 1:SP - 1, :]),
                     xs[:, 2:SP, :])
    k0 = r0 - WP - 1
    mhc = jnp.maximum(
        jnp.maximum(mw[:, k0:k0 + SH, :], mw[:, k0 + WP:k0 + WP + SH, :]),
        mw[:, k0 + 2 * WP:k0 + 2 * WP + SH, :])
    pooled = jnp.maximum(jnp.maximum(mhc[0:DB], mhc[1:1 + DB]),
                         mhc[2:2 + DB])
    y3 = jnp.maximum(
        jnp.dot(pooled.reshape(M, C), w3_ref[...],
                preferred_element_type=jnp.float32) + b3_ref[...], 0.0)

    out = jnp.concatenate([y0, y1, y2, y3], axis=-1)
    out = out.reshape(DB, H, WP, out.shape[-1])[:, :, 1:1 + W, :]
    o_ref[...] = out.astype(o_ref.dtype)


def kernel(x,
           b0_w, b0_s, b0_b,
           b1a_w, b1a_s, b1a_b,
           b1b_w, b1b_s, b1b_b,
           b2a_w, b2a_s, b2a_b,
           b2b_w, b2b_s, b2b_b,
           b3_w, b3_s, b3_b):
    n, c, d, h, w = x.shape
    bf = jnp.bfloat16
    dp, wp = d + 2, w + 2
    sp = (h + 4) * wp

    xt = jnp.transpose(x, (0, 2, 3, 4, 1)).astype(bf)
    xf = jnp.pad(xt, ((0, 0), (1, 1), (2, 2), (1, 1), (0, 0)),
                 constant_values=-jnp.inf)
    xf = xf.reshape(n, dp, sp, c)

    ii = jnp.arange(sp)
    hh = ii // wp - 1
    ww = ii % wp
    smask = ((hh >= 1) & (hh <= h) & (ww >= 1) & (ww <= w)
             ).astype(jnp.float32).reshape(sp, 1)

    c0 = b0_w.shape[1]
    c1 = b1a_w.shape[1]
    c2 = b2a_w.shape[1]
    c1b = b1b_w.shape[-1]
    c2b = b2b_w.shape[-1]
    c3 = b3_w.shape[1]
    couts = c0 + c1b + c2b + c3

    db = d // 2 if d % 2 == 0 else d

    w0f = (b0_w * b0_s[None, :]).astype(bf)
    w12 = jnp.concatenate([b1a_w * b1a_s[None, :],
                           b2a_w * b2a_s[None, :]], axis=1).astype(bf)
    b12 = jnp.concatenate([b1a_b, b2a_b]).reshape(1, c1 + c2)
    w1f = (b1b_w * b1b_s).reshape(3, 9 * c1, c1b).astype(bf)
    w2f = (b2b_w * b2b_s).reshape(27 * c2, c2b).astype(bf)
    w3f = (b3_w * b3_s[None, :]).astype(bf)

    out = pl.pallas_call(
        functools.partial(_mixed_kernel, D=d, H=h, W=w, C1=c1, C2=c2, SP=sp),
        out_shape=jax.ShapeDtypeStruct((n, d, h, w, couts), jnp.float32),
        grid_spec=pltpu.PrefetchScalarGridSpec(
            num_scalar_prefetch=0,
            grid=(n, d // db),
            in_specs=[
                pl.BlockSpec((pl.Squeezed(), dp, sp, c),
                             lambda ni, di: (ni, 0, 0, 0)),
                pl.BlockSpec((c, c0), lambda ni, di: (0, 0)),
                pl.BlockSpec((1, c0), lambda ni, di: (0, 0)),
                pl.BlockSpec((c, c1 + c2), lambda ni, di: (0, 0)),
                pl.BlockSpec((1, c1 + c2), lambda ni, di: (0, 0)),
                pl.BlockSpec((3, 9 * c1, c1b), lambda ni, di: (0, 0, 0)),
                pl.BlockSpec((1, c1b), lambda ni, di: (0, 0)),
                pl.BlockSpec((27 * c2, c2b), lambda ni, di: (0, 0)),
                pl.BlockSpec((1, c2b), lambda ni, di: (0, 0)),
                pl.BlockSpec((c, c3), lambda ni, di: (0, 0)),
                pl.BlockSpec((1, c3), lambda ni, di: (0, 0)),
                pl.BlockSpec((sp, 1), lambda ni, di: (0, 0)),
            ],
            out_specs=pl.BlockSpec((pl.Squeezed(), db, h, w, couts),
                                   lambda ni, di: (ni, di, 0, 0, 0)),
        ),
        compiler_params=pltpu.CompilerParams(
            dimension_semantics=("parallel", "parallel"),
            vmem_limit_bytes=60 * 1024 * 1024,
        ),
    )(xf, w0f, b0_b.reshape(1, c0), w12, b12,
      w1f, b1b_b.reshape(1, c1b), w2f, b2b_b.reshape(1, c2b),
      w3f, b3_b.reshape(1, c3), smask)
    return jnp.transpose(out, (0, 4, 1, 2, 3))


# speedup vs baseline: 1.5439x; 1.0613x over previous
import functools

import jax
import jax.numpy as jnp
from jax.experimental import pallas as pl
from jax.experimental.pallas import tpu as pltpu


def _mixed_kernel(xp_ref, w0_ref, b0_ref, w12_ref, b12_ref,
                  w1_ref, b1_ref, w3_ref, b3_ref, sm_ref,
                  o_ref, *, D, H, W, C1, C2, SP):
    WP = W + 2
    SH = H * WP
    r0 = 2 * WP
    DB = o_ref.shape[0]
    DS = DB + 2
    M = DB * SH
    d0 = pl.multiple_of(pl.program_id(1) * DB, DB)
    xs = xp_ref[pl.ds(d0, DS)]
    C = xs.shape[-1]

    dd = d0 + jax.lax.broadcasted_iota(jnp.int32, (DS, 1, 1), 0)
    svalid = (sm_ref[...] != 0)[None, :, :]
    interior = (dd >= 1) & (dd <= D) & svalid

    hid = jnp.dot(xs.reshape(DS * SP, C), w12_ref[...],
                  preferred_element_type=jnp.float32)
    hid = jnp.maximum(hid + b12_ref[...], 0.0)
    hid = jnp.where(interior.reshape(DS * SP, 1), hid, 0.0)
    hs = hid.astype(jnp.bfloat16).reshape(DS, SP, C1 + C2)

    xin = xs[1:1 + DB, r0:r0 + SH, :].reshape(M, C)
    y0 = jnp.maximum(
        jnp.dot(xin, w0_ref[...], preferred_element_type=jnp.float32)
        + b0_ref[...], 0.0)

    acc = jnp.zeros((M, w1_ref.shape[-1]), jnp.float32)
    for kd in range(3):
        taps = [hs[kd:kd + DB,
                   r0 + (kh - 1) * WP + kw - 1:
                   r0 + (kh - 1) * WP + kw - 1 + SH, :]
                for kh in range(3) for kw in range(3)]
        wide = jnp.concatenate(taps, axis=-1).reshape(M, 9 * (C1 + C2))
        acc = acc + jnp.dot(wide, w1_ref[kd],
                            preferred_element_type=jnp.float32)
    y12 = jnp.maximum(acc + b1_ref[...], 0.0)

    mw = jnp.maximum(jnp.maximum(xs[:, 0:SP - 2, :], xs[:, 1:SP - 1, :]),
                     xs[:, 2:SP, :])
    k0 = r0 - WP - 1
    mhc = jnp.maximum(
        jnp.maximum(mw[:, k0:k0 + SH, :], mw[:, k0 + WP:k0 + WP + SH, :]),
        mw[:, k0 + 2 * WP:k0 + 2 * WP + SH, :])
    pooled = jnp.maximum(jnp.maximum(mhc[0:DB], mhc[1:1 + DB]),
                         mhc[2:2 + DB])
    y3 = jnp.maximum(
        jnp.dot(pooled.reshape(M, C), w3_ref[...],
                preferred_element_type=jnp.float32) + b3_ref[...], 0.0)

    out = jnp.concatenate([y0, y12, y3], axis=-1)
    out = out.reshape(DB, H, WP, out.shape[-1])[:, :, 1:1 + W, :]
    o_ref[...] = out.astype(o_ref.dtype)


def kernel(x,
           b0_w, b0_s, b0_b,
           b1a_w, b1a_s, b1a_b,
           b1b_w, b1b_s, b1b_b,
           b2a_w, b2a_s, b2a_b,
           b2b_w, b2b_s, b2b_b,
           b3_w, b3_s, b3_b):
    n, c, d, h, w = x.shape
    bf = jnp.bfloat16
    dp, wp = d + 2, w + 2
    sp = (h + 4) * wp

    xt = jnp.transpose(x, (0, 2, 3, 4, 1)).astype(bf)
    xf = jnp.pad(xt, ((0, 0), (1, 1), (2, 2), (1, 1), (0, 0)),
                 constant_values=-jnp.inf)
    xf = xf.reshape(n, dp, sp, c)

    ii = jnp.arange(sp)
    hh = ii // wp - 1
    ww = ii % wp
    smask = ((hh >= 1) & (hh <= h) & (ww >= 1) & (ww <= w)
             ).astype(jnp.float32).reshape(sp, 1)

    c0 = b0_w.shape[1]
    c1 = b1a_w.shape[1]
    c2 = b2a_w.shape[1]
    c1b = b1b_w.shape[-1]
    c2b = b2b_w.shape[-1]
    c3 = b3_w.shape[1]
    couts = c0 + c1b + c2b + c3

    db = d // 2 if d % 2 == 0 else d

    w0f = (b0_w * b0_s[None, :]).astype(bf)
    w12 = jnp.concatenate([b1a_w * b1a_s[None, :],
                           b2a_w * b2a_s[None, :]], axis=1).astype(bf)
    b12 = jnp.concatenate([b1a_b, b2a_b]).reshape(1, c1 + c2)
    w12b = jnp.zeros((3, 3, 3, c1 + c2, c1b + c2b), jnp.float32)
    w12b = w12b.at[:, :, :, :c1, :c1b].set(b1b_w * b1b_s)
    w12b = w12b.at[:, :, :, c1:, c1b:].set(b2b_w * b2b_s)
    w1f = w12b.reshape(3, 9 * (c1 + c2), c1b + c2b).astype(bf)
    b12bf = jnp.concatenate([b1b_b, b2b_b]).reshape(1, c1b + c2b)
    w3f = (b3_w * b3_s[None, :]).astype(bf)

    out = pl.pallas_call(
        functools.partial(_mixed_kernel, D=d, H=h, W=w, C1=c1, C2=c2, SP=sp),
        out_shape=jax.ShapeDtypeStruct((n, d, h, w, couts), jnp.float32),
        grid_spec=pltpu.PrefetchScalarGridSpec(
            num_scalar_prefetch=0,
            grid=(n, d // db),
            in_specs=[
                pl.BlockSpec((pl.Squeezed(), dp, sp, c),
                             lambda ni, di: (ni, 0, 0, 0)),
                pl.BlockSpec((c, c0), lambda ni, di: (0, 0)),
                pl.BlockSpec((1, c0), lambda ni, di: (0, 0)),
                pl.BlockSpec((c, c1 + c2), lambda ni, di: (0, 0)),
                pl.BlockSpec((1, c1 + c2), lambda ni, di: (0, 0)),
                pl.BlockSpec((3, 9 * (c1 + c2), c1b + c2b),
                             lambda ni, di: (0, 0, 0)),
                pl.BlockSpec((1, c1b + c2b), lambda ni, di: (0, 0)),
                pl.BlockSpec((c, c3), lambda ni, di: (0, 0)),
                pl.BlockSpec((1, c3), lambda ni, di: (0, 0)),
                pl.BlockSpec((sp, 1), lambda ni, di: (0, 0)),
            ],
            out_specs=pl.BlockSpec((pl.Squeezed(), db, h, w, couts),
                                   lambda ni, di: (ni, di, 0, 0, 0)),
        ),
        compiler_params=pltpu.CompilerParams(
            dimension_semantics=("parallel", "parallel"),
            vmem_limit_bytes=60 * 1024 * 1024,
        ),
    )(xf, w0f, b0_b.reshape(1, c0), w12, b12,
      w1f, b12bf, w3f, b3_b.reshape(1, c3), smask)
    return jnp.transpose(out, (0, 4, 1, 2, 3))
